# Initial kernel scaffold; baseline (speedup 1.0000x reference)
import jax
import jax.numpy as jnp
from jax.experimental import pallas as pl


def kernel(base_weight):
    raise NotImplementedError("write your pallas kernel here")


# speedup vs baseline: 26.7422x; 26.7422x over previous
import functools

import jax
import jax.numpy as jnp
from jax import lax
from jax.experimental import pallas as pl
from jax.experimental.pallas import tpu as pltpu
from jax.experimental.pallas import tpu_sc as plsc

N = 16777216
K = 65536
NW = 32
PER_TILE = N // NW
CH = 8192
NCH = PER_TILE // CH
B1 = 65536
B2 = 32768
ROWS = 2048
COLS = N // ROWS
GR = 16
BROWS = ROWS // GR

_MESH = plsc.VectorSubcoreMesh(
    core_axis_name="c", subcore_axis_name="s", num_cores=2, num_subcores=16)
_SC_PARAMS = pltpu.CompilerParams(needs_layout_passes=False)


def _coarse_hist_body(w_hbm, out_hbm, hist_v, buf_v):
    wid = lax.axis_index("s") * 2 + lax.axis_index("c")

    @pl.loop(0, B1 // 16)
    def _zero(i):
        hist_v[pl.ds(i * 16, 16)] = jnp.zeros((16,), jnp.int32)

    ones = jnp.ones((16,), jnp.int32)
    base = wid * PER_TILE

    @pl.loop(0, NCH)
    def _chunk(ci):
        pltpu.sync_copy(w_hbm.at[pl.ds(base + ci * CH, CH)], buf_v)

        @pl.loop(0, CH // 16)
        def _vec(j):
            bits = plsc.bitcast(buf_v[pl.ds(j * 16, 16)], jnp.int32)
            babs = jnp.bitwise_and(bits, jnp.int32(0x7FFFFFFF))
            bkt = lax.shift_right_logical(babs, 15)
            plsc.addupdate_scatter(hist_v, [bkt], ones)

    pltpu.sync_copy(hist_v, out_hbm.at[wid])


_coarse_hist = functools.partial(
    pl.kernel,
    out_type=jax.ShapeDtypeStruct((NW, B1), jnp.int32),
    mesh=_MESH,
    scratch_types=[
        pltpu.VMEM((B1,), jnp.int32),
        pltpu.VMEM((CH,), jnp.float32),
    ],
    compiler_params=_SC_PARAMS,
)(_coarse_hist_body)


def _fine_hist_body(w_hbm, params_hbm, out_hbm, hist_v, buf_v, b_v):
    wid = lax.axis_index("s") * 2 + lax.axis_index("c")
    pltpu.sync_copy(params_hbm.at[pl.ds(0, 16)], b_v)
    bvec = b_v[pl.ds(0, 16)]

    @pl.loop(0, B2 // 16)
    def _zero(i):
        hist_v[pl.ds(i * 16, 16)] = jnp.zeros((16,), jnp.int32)

    ones = jnp.ones((16,), jnp.int32)
    base = wid * PER_TILE

    @pl.loop(0, NCH)
    def _chunk(ci):
        pltpu.sync_copy(w_hbm.at[pl.ds(base + ci * CH, CH)], buf_v)

        @pl.loop(0, CH // 16)
        def _vec(j):
            bits = plsc.bitcast(buf_v[pl.ds(j * 16, 16)], jnp.int32)
            babs = jnp.bitwise_and(bits, jnp.int32(0x7FFFFFFF))
            bkt = lax.shift_right_logical(babs, 15)
            low = jnp.bitwise_and(babs, jnp.int32(0x7FFF))
            plsc.addupdate_scatter(hist_v, [low], ones, mask=bkt == bvec)

    pltpu.sync_copy(hist_v, out_hbm.at[wid])


_fine_hist = functools.partial(
    pl.kernel,
    out_type=jax.ShapeDtypeStruct((NW, B2), jnp.int32),
    mesh=_MESH,
    scratch_types=[
        pltpu.VMEM((B2,), jnp.int32),
        pltpu.VMEM((CH,), jnp.float32),
        pltpu.VMEM((16,), jnp.int32),
    ],
    compiler_params=_SC_PARAMS,
)(_fine_hist_body)


def _suffix_rows(x):
    s = x
    sh = 1
    while sh < x.shape[0]:
        s = s + jnp.concatenate(
            [s[sh:], jnp.zeros((sh, 128), s.dtype)], axis=0)
        sh *= 2
    return s


def _suffix_lanes(x):
    s = x
    sh = 1
    while sh < x.shape[1]:
        s = s + jnp.concatenate(
            [s[:, sh:], jnp.zeros((1, sh), s.dtype)], axis=1)
        sh *= 2
    return s


def _pick(h, kk):
    nrows = h.shape[0]
    rt = jnp.broadcast_to(jnp.sum(h, axis=1, keepdims=True), (nrows, 128))
    big = _suffix_rows(rt)
    ri = lax.broadcasted_iota(jnp.int32, (nrows, 128), 0)
    br = jnp.max(jnp.where(big >= kk, ri, -1))
    hrow = jnp.sum(jnp.where(ri == br, h, 0), axis=0, keepdims=True)
    above_rows = jnp.max(jnp.where(ri == br, big - rt, 0))
    s = above_rows + _suffix_lanes(hrow)
    li = lax.broadcasted_iota(jnp.int32, (1, 128), 1)
    bl = jnp.max(jnp.where(s >= kk, li, -1))
    strictly_above = jnp.max(jnp.where(li == bl, s - hrow, 0))
    in_bucket = jnp.max(jnp.where(li == bl, hrow, 0))
    return br * 128 + bl, kk - strictly_above, in_bucket


def _pick_body(h_ref, out_ref):
    h = jnp.sum(h_ref[...], axis=0)
    b, kp, _ = _pick(h, K)
    ri = lax.broadcasted_iota(jnp.int32, (8, 128), 0)
    out_ref[...] = jnp.where(ri == 0, b, jnp.where(ri == 1, kp, 0))


def _mask_body(params_ref, h2_ref, w_ref, o_ref, t_smem):
    @pl.when(pl.program_id(0) == 0)
    def _():
        h2 = jnp.sum(h2_ref[...], axis=0)
        b = params_ref[0, 0]
        kp = params_ref[1, 0]
        t, r_need, c_eq = _pick(h2, kp)
        t_smem[0] = jnp.bitwise_or(lax.shift_left(b, 15), t)
        t_smem[1] = r_need
        t_smem[2] = c_eq
        t_smem[3] = 0

    thr = t_smem[0]
    r_need = t_smem[1]
    c_eq = t_smem[2]
    seen = t_smem[3]
    bits = lax.bitcast_convert_type(w_ref[...], jnp.int32)
    babs = jnp.bitwise_and(bits, jnp.int32(0x7FFFFFFF))
    eq = (babs == thr).astype(jnp.int32)
    eq_blk = jnp.sum(eq)

    @pl.when((eq_blk == 0) | (r_need == c_eq))
    def _():
        o_ref[...] = jnp.where(babs >= thr, 1.0, 0.0).astype(jnp.float32)

    @pl.when((eq_blk != 0) & (r_need != c_eq))
    def _():
        lane_in = eq
        sh = 1
        while sh < COLS:
            lane_in = lane_in + jnp.concatenate(
                [jnp.zeros((BROWS, sh), jnp.int32), lane_in[:, :-sh]], axis=1)
            sh *= 2
        lane_excl = lane_in - eq
        rc = jnp.sum(eq, axis=1, keepdims=True)
        row_in = rc
        sh = 1
        while sh < BROWS:
            row_in = row_in + jnp.concatenate(
                [jnp.zeros((sh, 1), jnp.int32), row_in[:-sh]], axis=0)
            sh *= 2
        rank = seen + (row_in - rc) + lane_excl
        keep = (babs > thr) | ((eq == 1) & (rank < r_need))
        o_ref[...] = jnp.where(keep, 1.0, 0.0).astype(jnp.float32)

    t_smem[3] = seen + eq_blk


def kernel(base_weight):
    hists1 = _coarse_hist(base_weight)
    params = pl.pallas_call(
        _pick_body,
        out_shape=jax.ShapeDtypeStruct((8, 128), jnp.int32),
    )(hists1.reshape(NW, 512, 128))
    hists2 = _fine_hist(base_weight, params.reshape(1024))
    mask2d = pl.pallas_call(
        _mask_body,
        grid=(GR,),
        in_specs=[
            pl.BlockSpec(memory_space=pltpu.SMEM),
            pl.BlockSpec((NW, 256, 128), lambda i: (0, 0, 0)),
            pl.BlockSpec((BROWS, COLS), lambda i: (i, 0)),
        ],
        out_specs=pl.BlockSpec((BROWS, COLS), lambda i: (i, 0)),
        out_shape=jax.ShapeDtypeStruct((ROWS, COLS), jnp.float32),
        scratch_shapes=[pltpu.SMEM((4,), jnp.int32)],
    )(params, hists2.reshape(NW, 256, 128), base_weight.reshape(ROWS, COLS))
    return mask2d.reshape(N)


# speedup vs baseline: 140.2768x; 5.2455x over previous
import functools

import jax
import jax.numpy as jnp
from jax import lax
from jax.experimental import pallas as pl
from jax.experimental.pallas import tpu as pltpu
from jax.experimental.pallas import tpu_sc as plsc

N = 16777216
K = 65536
NW = 32
PER_TILE = N // NW
CH = 8192
NCH = PER_TILE // CH
B1 = 65536
B2 = 32768
ROWS = N // 128
GR = 32
BROWS = ROWS // GR

_MESH = plsc.VectorSubcoreMesh(
    core_axis_name="c", subcore_axis_name="s", num_cores=2, num_subcores=16)
_SC_PARAMS = pltpu.CompilerParams(needs_layout_passes=False)


def _coarse_hist_body(w_hbm, out_hbm, hist_v, buf0, buf1, sem0, sem1):
    wid = lax.axis_index("s") * 2 + lax.axis_index("c")

    @functools.partial(plsc.parallel_loop, 0, B1 // 128, unroll=4)
    def _zero(r):
        for j in range(8):
            hist_v[r, pl.ds(j * 16, 16)] = jnp.zeros((16,), jnp.int32)

    ones = jnp.ones((16,), jnp.int32)
    base = wid * PER_TILE
    bufs = (buf0, buf1)
    sems = (sem0, sem1)
    pltpu.async_copy(w_hbm.at[pl.ds(base, CH)], buf0, sem0)
    pltpu.async_copy(w_hbm.at[pl.ds(base + CH, CH)], buf1, sem1)

    @pl.loop(0, NCH, step=2)
    def _chunks(ci):
        for b in range(2):
            buf, sem = bufs[b], sems[b]
            pltpu.make_async_copy(w_hbm.at[pl.ds(0, CH)], buf, sem).wait()

            @functools.partial(plsc.parallel_loop, 0, CH // 16, unroll=8)
            def _vec(j):
                bits = plsc.bitcast(buf[pl.ds(j * 16, 16)], jnp.int32)
                babs = jnp.bitwise_and(bits, jnp.int32(0x7FFFFFFF))
                bkt = lax.shift_right_logical(babs, 15)
                plsc.addupdate_scatter(
                    hist_v,
                    [lax.shift_right_logical(bkt, 7),
                     jnp.bitwise_and(bkt, jnp.int32(127))],
                    ones)

            nxt = ci + 2 + b

            @pl.when(nxt < NCH)
            def _():
                pltpu.async_copy(
                    w_hbm.at[pl.ds(base + nxt * CH, CH)], buf, sem)

    pltpu.sync_copy(hist_v, out_hbm.at[pl.ds(wid * (B1 // 128), B1 // 128)])


_coarse_hist = functools.partial(
    pl.kernel,
    out_type=jax.ShapeDtypeStruct((NW * (B1 // 128), 128), jnp.int32),
    mesh=_MESH,
    scratch_types=[
        pltpu.VMEM((B1 // 128, 128), jnp.int32),
        pltpu.VMEM((CH,), jnp.float32),
        pltpu.VMEM((CH,), jnp.float32),
        pltpu.SemaphoreType.DMA,
        pltpu.SemaphoreType.DMA,
    ],
    compiler_params=_SC_PARAMS,
)(_coarse_hist_body)


def _fine_hist_body(w_hbm, params_hbm, out_hbm, hist_v, buf0, buf1, b_v,
                    sem0, sem1):
    wid = lax.axis_index("s") * 2 + lax.axis_index("c")
    pltpu.sync_copy(params_hbm.at[pl.ds(0, 16)], b_v)
    bvec = b_v[pl.ds(0, 16)]

    @functools.partial(plsc.parallel_loop, 0, B2 // 128, unroll=4)
    def _zero(r):
        for j in range(8):
            hist_v[r, pl.ds(j * 16, 16)] = jnp.zeros((16,), jnp.int32)

    ones = jnp.ones((16,), jnp.int32)
    base = wid * PER_TILE
    bufs = (buf0, buf1)
    sems = (sem0, sem1)
    pltpu.async_copy(w_hbm.at[pl.ds(base, CH)], buf0, sem0)
    pltpu.async_copy(w_hbm.at[pl.ds(base + CH, CH)], buf1, sem1)

    @pl.loop(0, NCH, step=2)
    def _chunks(ci):
        for b in range(2):
            buf, sem = bufs[b], sems[b]
            pltpu.make_async_copy(w_hbm.at[pl.ds(0, CH)], buf, sem).wait()

            @functools.partial(plsc.parallel_loop, 0, CH // 16, unroll=8)
            def _vec(j):
                bits = plsc.bitcast(buf[pl.ds(j * 16, 16)], jnp.int32)
                babs = jnp.bitwise_and(bits, jnp.int32(0x7FFFFFFF))
                bkt = lax.shift_right_logical(babs, 15)
                low = jnp.bitwise_and(babs, jnp.int32(0x7FFF))
                plsc.addupdate_scatter(
                    hist_v,
                    [lax.shift_right_logical(low, 7),
                     jnp.bitwise_and(low, jnp.int32(127))],
                    ones, mask=bkt == bvec)

            nxt = ci + 2 + b

            @pl.when(nxt < NCH)
            def _():
                pltpu.async_copy(
                    w_hbm.at[pl.ds(base + nxt * CH, CH)], buf, sem)

    pltpu.sync_copy(hist_v, out_hbm.at[pl.ds(wid * (B2 // 128), B2 // 128)])


_fine_hist = functools.partial(
    pl.kernel,
    out_type=jax.ShapeDtypeStruct((NW * (B2 // 128), 128), jnp.int32),
    mesh=_MESH,
    scratch_types=[
        pltpu.VMEM((B2 // 128, 128), jnp.int32),
        pltpu.VMEM((CH,), jnp.float32),
        pltpu.VMEM((CH,), jnp.float32),
        pltpu.VMEM((16,), jnp.int32),
        pltpu.SemaphoreType.DMA,
        pltpu.SemaphoreType.DMA,
    ],
    compiler_params=_SC_PARAMS,
)(_fine_hist_body)


def _suffix_rows(x):
    s = x
    sh = 1
    while sh < x.shape[0]:
        s = s + jnp.concatenate(
            [s[sh:], jnp.zeros((sh, 128), s.dtype)], axis=0)
        sh *= 2
    return s


def _suffix_lanes(x):
    s = x
    sh = 1
    while sh < x.shape[1]:
        s = s + jnp.concatenate(
            [s[:, sh:], jnp.zeros((1, sh), s.dtype)], axis=1)
        sh *= 2
    return s


def _pick(h, kk):
    nrows = h.shape[0]
    rt = jnp.broadcast_to(jnp.sum(h, axis=1, keepdims=True), (nrows, 128))
    big = _suffix_rows(rt)
    ri = lax.broadcasted_iota(jnp.int32, (nrows, 128), 0)
    br = jnp.max(jnp.where(big >= kk, ri, -1))
    hrow = jnp.sum(jnp.where(ri == br, h, 0), axis=0, keepdims=True)
    above_rows = jnp.max(jnp.where(ri == br, big - rt, 0))
    s = above_rows + _suffix_lanes(hrow)
    li = lax.broadcasted_iota(jnp.int32, (1, 128), 1)
    bl = jnp.max(jnp.where(s >= kk, li, -1))
    strictly_above = jnp.max(jnp.where(li == bl, s - hrow, 0))
    in_bucket = jnp.max(jnp.where(li == bl, hrow, 0))
    return br * 128 + bl, kk - strictly_above, in_bucket


def _merge(h_ref, nbkt):
    h = h_ref[pl.ds(0, nbkt // 128), :]
    for wd in range(1, NW):
        h = h + h_ref[pl.ds(wd * (nbkt // 128), nbkt // 128), :]
    return h


def _pick_body(h_ref, out_ref):
    b, kp, _ = _pick(_merge(h_ref, B1), K)
    ri = lax.broadcasted_iota(jnp.int32, (8, 128), 0)
    out_ref[...] = jnp.where(ri == 0, b, jnp.where(ri == 1, kp, 0))


def _mask_body(params_ref, h2_ref, w_ref, o_ref, t_smem):
    @pl.when(pl.program_id(0) == 0)
    def _():
        b = params_ref[0, 0]
        kp = params_ref[1, 0]
        t, r_need, c_eq = _pick(_merge(h2_ref, B2), kp)
        t_smem[0] = jnp.bitwise_or(lax.shift_left(b, 15), t)
        t_smem[1] = r_need
        t_smem[2] = c_eq
        t_smem[3] = 0

    thr = t_smem[0]
    r_need = t_smem[1]
    c_eq = t_smem[2]
    seen = t_smem[3]
    bits = lax.bitcast_convert_type(w_ref[...], jnp.int32)
    babs = jnp.bitwise_and(bits, jnp.int32(0x7FFFFFFF))
    eq = (babs == thr).astype(jnp.int32)
    eq_blk = jnp.sum(eq)

    @pl.when((eq_blk == 0) | (r_need == c_eq))
    def _():
        o_ref[...] = jnp.where(babs >= thr, 1.0, 0.0).astype(jnp.float32)

    @pl.when((eq_blk != 0) & (r_need != c_eq))
    def _():
        lane_in = eq
        sh = 1
        while sh < 128:
            lane_in = lane_in + jnp.concatenate(
                [jnp.zeros((BROWS, sh), jnp.int32), lane_in[:, :-sh]], axis=1)
            sh *= 2
        lane_excl = lane_in - eq
        rc = jnp.sum(eq, axis=1, keepdims=True)
        row_in = rc
        sh = 1
        while sh < BROWS:
            row_in = row_in + jnp.concatenate(
                [jnp.zeros((sh, 1), jnp.int32), row_in[:-sh]], axis=0)
            sh *= 2
        rank = seen + (row_in - rc) + lane_excl
        keep = (babs > thr) | ((eq == 1) & (rank < r_need))
        o_ref[...] = jnp.where(keep, 1.0, 0.0).astype(jnp.float32)

    t_smem[3] = seen + eq_blk


def kernel(base_weight):
    hists1 = _coarse_hist(base_weight)
    params = pl.pallas_call(
        _pick_body,
        out_shape=jax.ShapeDtypeStruct((8, 128), jnp.int32),
    )(hists1)
    hists2 = _fine_hist(base_weight, params.reshape(1024))
    mask2d = pl.pallas_call(
        _mask_body,
        grid=(GR,),
        in_specs=[
            pl.BlockSpec(memory_space=pltpu.SMEM),
            pl.BlockSpec((NW * (B2 // 128), 128), lambda i: (0, 0)),
            pl.BlockSpec((BROWS, 128), lambda i: (i, 0)),
        ],
        out_specs=pl.BlockSpec((BROWS, 128), lambda i: (i, 0)),
        out_shape=jax.ShapeDtypeStruct((ROWS, 128), jnp.float32),
        scratch_shapes=[pltpu.SMEM((4,), jnp.int32)],
    )(params, hists2, base_weight.reshape(ROWS, 128))
    return mask2d.reshape(N)
